# baseline (speedup 1.0000x reference)
import functools

import jax
import jax.numpy as jnp
from jax import lax
from jax.experimental import pallas as pl
from jax.experimental.pallas import tpu as pltpu
from jax.experimental.pallas import tpu_sc as plsc

N = 262144
NC = 2
NS = 16
L = 16
NW = NC * NS
EPT = N // NS
UPT = N // NW
SHIFT = 16
SPLIT = 16281
NBREAL = 16512
NBALLOC = 18432
BPT = NBALLOC // NS
DUMMY0 = NBALLOC - NS
CHUNK = 128
MAXB = EPT + 2 * CHUNK
NWAVES = 16
VPW = EPT // L // NWAVES
MAXCPW = 9
DENOM = 34359607296.0

_mesh = plsc.VectorSubcoreMesh(core_axis_name="c", subcore_axis_name="s")
_params = pltpu.CompilerParams(needs_layout_passes=False)


@functools.partial(
    pl.kernel,
    out_type=jax.ShapeDtypeStruct((4 * L,), jnp.float32),
    mesh=_mesh,
    compiler_params=_params,
    scratch_types=[
        pltpu.VMEM((EPT,), jnp.float32),
        pltpu.VMEM((EPT,), jnp.float32),
        pltpu.VMEM((UPT,), jnp.float32),
        pltpu.VMEM((MAXB,), jnp.int32),
        pltpu.VMEM((MAXB,), jnp.float32),
        pltpu.VMEM((CHUNK,), jnp.int32),
        pltpu.VMEM((BPT,), jnp.int32),
        pltpu.VMEM((BPT,), jnp.float32),
        pltpu.VMEM((BPT,), jnp.int32),
        pltpu.VMEM((BPT,), jnp.float32),
        pltpu.VMEM((L,), jnp.int32),
        pltpu.VMEM((L,), jnp.int32),
        pltpu.VMEM((L,), jnp.float32),
        pltpu.VMEM((L,), jnp.float32),
        pltpu.VMEM((NS * L,), jnp.int32),
        pltpu.VMEM((NS * L,), jnp.int32),
        pltpu.VMEM((NS * L,), jnp.float32),
        pltpu.VMEM((NS * L,), jnp.float32),
        pltpu.VMEM((2 * L,), jnp.float32),
        pltpu.VMEM_SHARED((NBALLOC,), jnp.int32),
        pltpu.VMEM_SHARED((NBALLOC,), jnp.float32),
        pltpu.VMEM_SHARED((NS * L,), jnp.int32),
        pltpu.VMEM_SHARED((NS * L,), jnp.int32),
        pltpu.VMEM_SHARED((NS * L,), jnp.float32),
        pltpu.VMEM_SHARED((NS * L,), jnp.float32),
        pltpu.SemaphoreType.DMA,
        pltpu.SemaphoreType.DMA,
    ],
)
def _fused(pred_h, targ_h, unc_h, out_h,
           pred_v, targ_v, unc_v, key_v, val_v, ones_v,
           zi_v, zf_v, ca_v, sa_v, ti_v, tz_v, tf_v, tu_v,
           tota_v, zca_v, para_v, upa_v, res_v,
           cnt_s, sum_s, tot_s, zc_s, par_s, unc_s,
           scat_sem, in_sem):
    c = lax.axis_index("c")
    s = lax.axis_index("s")
    wid = c * NS + s

    in_descs = [
        pltpu.async_copy(pred_h.at[pl.ds(s * EPT, EPT)], pred_v, in_sem),
        pltpu.async_copy(targ_h.at[pl.ds(s * EPT, EPT)], targ_v, in_sem),
        pltpu.async_copy(unc_h.at[pl.ds(wid * UPT, UPT)], unc_v, in_sem),
    ]

    def zfill(i, _):
        zi_v[pl.ds(i * L, L)] = jnp.zeros((L,), jnp.int32)
        zf_v[pl.ds(i * L, L)] = jnp.zeros((L,), jnp.float32)
        return 0
    lax.fori_loop(0, BPT // L, zfill, 0)
    for k in range(CHUNK // L):
        ones_v[pl.ds(k * L, L)] = jnp.ones((L,), jnp.int32)
    pltpu.sync_copy(zi_v, cnt_s.at[pl.ds(s * BPT, BPT)])
    pltpu.sync_copy(zf_v, sum_s.at[pl.ds(s * BPT, BPT)])
    for d in in_descs:
        d.wait()

    shift = jnp.full((L,), SHIFT, jnp.int32)
    lo_k = jnp.full((L,), 1, jnp.int32) * (c * SPLIT)
    hi_k = jnp.full((L,), 1, jnp.int32) * (SPLIT + c * 65536)
    dummy_k = jnp.full((L,), DUMMY0, jnp.int32) + s

    plsc.subcore_barrier()

    def cwave(i, carry):
        off, zacc = carry
        p = pred_v[pl.ds(i * L, L)]
        t = targ_v[pl.ds(i * L, L)]
        sgn = 2.0 * t - 1.0
        v = jnp.maximum(1.0 - p * sgn, 0.0)
        bits = lax.bitcast_convert_type(v, jnp.int32)
        kraw = lax.shift_right_logical(bits, shift)
        own = jnp.logical_and(
            jnp.logical_and(kraw >= lo_k, kraw < hi_k), v > 0.0)
        klocal = kraw - lo_k
        plsc.store_compressed(key_v.at[pl.ds(off, L)], klocal, mask=own)
        plsc.store_compressed(val_v.at[pl.ds(off, L)], v, mask=own)
        off = off + jnp.sum(jnp.where(own, 1, 0))
        zacc = zacc + jnp.where(v <= 0.0, 1, 0)
        return off, zacc

    off = jnp.int32(0)
    zacc = jnp.zeros((L,), jnp.int32)
    che = jnp.int32(0)
    wave_info = []
    for w in range(NWAVES):
        off, zacc = lax.fori_loop(w * VPW, (w + 1) * VPW, cwave, (off, zacc))
        allm = jnp.ones((L,), jnp.bool_)
        for k in range(CHUNK // L):
            plsc.store_compressed(key_v.at[pl.ds(off + k * L, L)],
                                  dummy_k, mask=allm)
            plsc.store_compressed(val_v.at[pl.ds(off + k * L, L)],
                                  jnp.zeros((L,), jnp.float32), mask=allm)
        offp = pl.multiple_of(
            lax.div(off + (CHUNK - 1), jnp.int32(CHUNK)) * CHUNK, CHUNK)
        chs = che
        che = lax.div(offp, jnp.int32(CHUNK))
        descs = []
        for j in range(MAXCPW):
            ch = chs + j
            cho = pl.multiple_of(ch * CHUNK, CHUNK)

            @pl.when(ch < che)
            def _fire(cho=cho, descs=descs):
                descs.append(pltpu.async_copy(
                    val_v.at[pl.ds(cho, CHUNK)],
                    sum_s.at[key_v.at[pl.ds(cho, CHUNK)]],
                    scat_sem, add=True))
                descs.append(pltpu.async_copy(
                    ones_v, cnt_s.at[key_v.at[pl.ds(cho, CHUNK)]],
                    scat_sem, add=True))
        wave_info.append((chs, che, descs))
        off = offp
        if w >= 2:
            pchs, pche, pdescs = wave_info[w - 2]
            for j in range(len(pdescs) // 2):
                @pl.when(pchs + j < pche)
                def _drain(j=j, pdescs=pdescs):
                    pdescs[2 * j].wait()
                    pdescs[2 * j + 1].wait()

    def ubody(i, acc):
        return acc + unc_v[pl.ds(i * L, L)]
    uacc = lax.fori_loop(0, UPT // L, ubody, jnp.zeros((L,), jnp.float32))

    for w in (NWAVES - 2, NWAVES - 1):
        pchs, pche, pdescs = wave_info[w]
        for j in range(len(pdescs) // 2):
            @pl.when(pchs + j < pche)
            def _drain(j=j, pdescs=pdescs):
                pdescs[2 * j].wait()
                pdescs[2 * j + 1].wait()

    plsc.subcore_barrier()

    pltpu.sync_copy(cnt_s.at[pl.ds(s * BPT, BPT)], ca_v)
    pltpu.sync_copy(sum_s.at[pl.ds(s * BPT, BPT)], sa_v)

    def totb(i, acc):
        return acc + ca_v[pl.ds(i * L, L)]
    tot = lax.fori_loop(0, BPT // L, totb, jnp.zeros((L,), jnp.int32))

    @pl.when(s == NS - 1)
    def _():
        ti_v[...] = tot - ca_v[pl.ds(BPT - L, L)]

    @pl.when(s != NS - 1)
    def _():
        ti_v[...] = tot

    tz_v[...] = zacc
    tu_v[...] = uacc
    pltpu.sync_copy(ti_v, tot_s.at[pl.ds(s * L, L)])
    pltpu.sync_copy(tz_v, zc_s.at[pl.ds(s * L, L)])
    pltpu.sync_copy(tu_v, unc_s.at[pl.ds(s * L, L)])
    plsc.subcore_barrier()

    pltpu.sync_copy(tot_s, tota_v)
    pltpu.sync_copy(zc_s, zca_v)

    zsum = jnp.int32(0)
    tsum = jnp.int32(0)
    base = jnp.int32(0)
    for t in range(NS):
        rowt = jnp.sum(tota_v[pl.ds(t * L, L)])
        zsum = zsum + jnp.sum(zca_v[pl.ds(t * L, L)])
        tsum = tsum + rowt
        base = base + jnp.where(jnp.int32(t) < s, rowt, jnp.int32(0))
    grand = jnp.where(c == 0, zsum, N - tsum)
    base = base + grand

    def scanb(i, carry):
        run, acc = carry
        cc = ca_v[pl.ds(i * L, L)]
        ss = sa_v[pl.ds(i * L, L)]
        cs = lax.cumsum(cc, axis=0)
        pexc = (run + cs - cc).astype(jnp.float32)
        mf = cc.astype(jnp.float32)
        acc = acc + ss * ((N - 1.0) - pexc - 0.5 * (mf - 1.0))
        run = run + jnp.sum(cc)
        return run, acc
    _, acc = lax.fori_loop(0, BPT // L, scanb,
                           (base, jnp.zeros((L,), jnp.float32)))
    tf_v[...] = acc
    pltpu.sync_copy(tf_v, par_s.at[pl.ds(s * L, L)])
    plsc.subcore_barrier()

    @pl.when(s == 0)
    def _():
        pltpu.sync_copy(par_s, para_v)
        pltpu.sync_copy(unc_s, upa_v)
        numer = jnp.zeros((L,), jnp.float32)
        usum = jnp.zeros((L,), jnp.float32)
        for t in range(NS):
            numer = numer + para_v[pl.ds(t * L, L)]
            usum = usum + upa_v[pl.ds(t * L, L)]
        nsc = jnp.sum(numer)
        usc = jnp.sum(usum)
        res_v[pl.ds(0, L)] = jnp.full((L,), 1.0, jnp.float32) * nsc
        res_v[pl.ds(L, L)] = jnp.full((L,), 1.0, jnp.float32) * usc
        pltpu.sync_copy(res_v, out_h.at[pl.ds(c * 2 * L, 2 * L)])


def kernel(pred, target, uncertainty_map):
    o = _fused(pred, target, uncertainty_map)
    numer = o[0] + o[2 * L]
    usum = o[L] + o[3 * L]
    return numer * (1.0 / DENOM) * (1.0 + usum * (1.0 / N))


# speedup vs baseline: 1.0195x; 1.0195x over previous
import functools

import jax
import jax.numpy as jnp
from jax import lax
from jax.experimental import pallas as pl
from jax.experimental.pallas import tpu as pltpu
from jax.experimental.pallas import tpu_sc as plsc

N = 262144
NC = 2
NS = 16
L = 16
NW = NC * NS
EPT = N // NS
UPT = N // NW
SHIFT = 16
SPLIT = 16281
NBREAL = 16512
NBALLOC = 18432
BPT = NBALLOC // NS
DUMMY0 = NBALLOC - NS
CHUNK = 128
MAXB = EPT + 2 * CHUNK
NWAVES = 16
VPW = EPT // L // NWAVES
MAXCPW = 9
DENOM = 34359607296.0

_mesh = plsc.VectorSubcoreMesh(core_axis_name="c", subcore_axis_name="s")
_params = pltpu.CompilerParams(needs_layout_passes=False)


@functools.partial(
    pl.kernel,
    out_type=jax.ShapeDtypeStruct((4 * L,), jnp.float32),
    mesh=_mesh,
    compiler_params=_params,
    scratch_types=[
        pltpu.VMEM((EPT,), jnp.float32),
        pltpu.VMEM((EPT,), jnp.float32),
        pltpu.VMEM((UPT,), jnp.float32),
        pltpu.VMEM((MAXB,), jnp.int32),
        pltpu.VMEM((MAXB,), jnp.float32),
        pltpu.VMEM((CHUNK,), jnp.int32),
        pltpu.VMEM((BPT,), jnp.int32),
        pltpu.VMEM((BPT,), jnp.float32),
        pltpu.VMEM((BPT,), jnp.int32),
        pltpu.VMEM((BPT,), jnp.float32),
        pltpu.VMEM((L,), jnp.int32),
        pltpu.VMEM((L,), jnp.int32),
        pltpu.VMEM((L,), jnp.float32),
        pltpu.VMEM((L,), jnp.float32),
        pltpu.VMEM((NS * L,), jnp.int32),
        pltpu.VMEM((NS * L,), jnp.int32),
        pltpu.VMEM((NS * L,), jnp.float32),
        pltpu.VMEM((NS * L,), jnp.float32),
        pltpu.VMEM((2 * L,), jnp.float32),
        pltpu.VMEM_SHARED((NBALLOC,), jnp.int32),
        pltpu.VMEM_SHARED((NBALLOC,), jnp.float32),
        pltpu.VMEM_SHARED((NS * L,), jnp.int32),
        pltpu.VMEM_SHARED((NS * L,), jnp.int32),
        pltpu.VMEM_SHARED((NS * L,), jnp.float32),
        pltpu.VMEM_SHARED((NS * L,), jnp.float32),
        pltpu.SemaphoreType.DMA,
        pltpu.SemaphoreType.DMA,
    ],
)
def _fused(pred_h, targ_h, unc_h, out_h,
           pred_v, targ_v, unc_v, key_v, val_v, ones_v,
           zi_v, zf_v, ca_v, sa_v, ti_v, tz_v, tf_v, tu_v,
           tota_v, zca_v, para_v, upa_v, res_v,
           cnt_s, sum_s, tot_s, zc_s, par_s, unc_s,
           scat_sem, in_sem):
    c = lax.axis_index("c")
    s = lax.axis_index("s")
    wid = c * NS + s

    in_descs = [
        pltpu.async_copy(pred_h.at[pl.ds(s * EPT, EPT)], pred_v, in_sem),
        pltpu.async_copy(targ_h.at[pl.ds(s * EPT, EPT)], targ_v, in_sem),
        pltpu.async_copy(unc_h.at[pl.ds(wid * UPT, UPT)], unc_v, in_sem),
    ]

    def zfill(i, _):
        zi_v[pl.ds(i * L, L)] = jnp.zeros((L,), jnp.int32)
        zf_v[pl.ds(i * L, L)] = jnp.zeros((L,), jnp.float32)
        return 0
    lax.fori_loop(0, BPT // L, zfill, 0)
    for k in range(CHUNK // L):
        ones_v[pl.ds(k * L, L)] = jnp.ones((L,), jnp.int32)
    pltpu.sync_copy(zi_v, cnt_s.at[pl.ds(s * BPT, BPT)])
    pltpu.sync_copy(zf_v, sum_s.at[pl.ds(s * BPT, BPT)])
    for d in in_descs:
        d.wait()

    shift = jnp.full((L,), SHIFT, jnp.int32)
    lo_k = jnp.full((L,), 1, jnp.int32) * (c * SPLIT)
    hi_k = jnp.full((L,), 1, jnp.int32) * (SPLIT + c * 65536)
    dummy_k = jnp.full((L,), DUMMY0, jnp.int32) + s

    plsc.subcore_barrier()

    def cwave(i, carry):
        off, zacc = carry
        p = pred_v[pl.ds(i * L, L)]
        t = targ_v[pl.ds(i * L, L)]
        sgn = 2.0 * t - 1.0
        v = jnp.maximum(1.0 - p * sgn, 0.0)
        bits = lax.bitcast_convert_type(v, jnp.int32)
        kraw = lax.shift_right_logical(bits, shift)
        own = jnp.logical_and(
            jnp.logical_and(kraw >= lo_k, kraw < hi_k), v > 0.0)
        klocal = kraw - lo_k
        plsc.store_compressed(key_v.at[pl.ds(off, L)], klocal, mask=own)
        plsc.store_compressed(val_v.at[pl.ds(off, L)], v, mask=own)
        off = off + plsc.all_reduce_population_count(own)[0]
        zacc = zacc + jnp.where(v <= 0.0, 1, 0)
        return off, zacc

    off = jnp.int32(0)
    zacc = jnp.zeros((L,), jnp.int32)
    che = jnp.int32(0)
    wave_info = []
    for w in range(NWAVES):
        off, zacc = lax.fori_loop(w * VPW, (w + 1) * VPW, cwave, (off, zacc))
        allm = jnp.ones((L,), jnp.bool_)
        for k in range(CHUNK // L):
            plsc.store_compressed(key_v.at[pl.ds(off + k * L, L)],
                                  dummy_k, mask=allm)
            plsc.store_compressed(val_v.at[pl.ds(off + k * L, L)],
                                  jnp.zeros((L,), jnp.float32), mask=allm)
        offp = pl.multiple_of(
            lax.div(off + (CHUNK - 1), jnp.int32(CHUNK)) * CHUNK, CHUNK)
        chs = che
        che = lax.div(offp, jnp.int32(CHUNK))
        descs = []
        for j in range(MAXCPW):
            ch = chs + j
            cho = pl.multiple_of(ch * CHUNK, CHUNK)

            @pl.when(ch < che)
            def _fire(cho=cho, descs=descs):
                descs.append(pltpu.async_copy(
                    val_v.at[pl.ds(cho, CHUNK)],
                    sum_s.at[key_v.at[pl.ds(cho, CHUNK)]],
                    scat_sem, add=True))
                descs.append(pltpu.async_copy(
                    ones_v, cnt_s.at[key_v.at[pl.ds(cho, CHUNK)]],
                    scat_sem, add=True))
        wave_info.append((chs, che, descs))
        off = offp
        if w >= 2:
            pchs, pche, pdescs = wave_info[w - 2]
            for j in range(len(pdescs) // 2):
                @pl.when(pchs + j < pche)
                def _drain(j=j, pdescs=pdescs):
                    pdescs[2 * j].wait()
                    pdescs[2 * j + 1].wait()

    def ubody(i, acc):
        return acc + unc_v[pl.ds(i * L, L)]
    uacc = lax.fori_loop(0, UPT // L, ubody, jnp.zeros((L,), jnp.float32))

    for w in (NWAVES - 2, NWAVES - 1):
        pchs, pche, pdescs = wave_info[w]
        for j in range(len(pdescs) // 2):
            @pl.when(pchs + j < pche)
            def _drain(j=j, pdescs=pdescs):
                pdescs[2 * j].wait()
                pdescs[2 * j + 1].wait()

    plsc.subcore_barrier()

    pltpu.sync_copy(cnt_s.at[pl.ds(s * BPT, BPT)], ca_v)
    pltpu.sync_copy(sum_s.at[pl.ds(s * BPT, BPT)], sa_v)

    def totb(i, acc):
        return acc + ca_v[pl.ds(i * L, L)]
    tot = lax.fori_loop(0, BPT // L, totb, jnp.zeros((L,), jnp.int32))

    @pl.when(s == NS - 1)
    def _():
        ti_v[...] = tot - ca_v[pl.ds(BPT - L, L)]

    @pl.when(s != NS - 1)
    def _():
        ti_v[...] = tot

    tz_v[...] = zacc
    tu_v[...] = uacc
    pltpu.sync_copy(ti_v, tot_s.at[pl.ds(s * L, L)])
    pltpu.sync_copy(tz_v, zc_s.at[pl.ds(s * L, L)])
    pltpu.sync_copy(tu_v, unc_s.at[pl.ds(s * L, L)])
    plsc.subcore_barrier()

    pltpu.sync_copy(tot_s, tota_v)
    pltpu.sync_copy(zc_s, zca_v)

    zsum = jnp.int32(0)
    tsum = jnp.int32(0)
    base = jnp.int32(0)
    for t in range(NS):
        rowt = jnp.sum(tota_v[pl.ds(t * L, L)])
        zsum = zsum + jnp.sum(zca_v[pl.ds(t * L, L)])
        tsum = tsum + rowt
        base = base + jnp.where(jnp.int32(t) < s, rowt, jnp.int32(0))
    grand = jnp.where(c == 0, zsum, N - tsum)
    base = base + grand

    def scanb(i, carry):
        run, acc = carry
        cc = ca_v[pl.ds(i * L, L)]
        ss = sa_v[pl.ds(i * L, L)]
        cs = lax.cumsum(cc, axis=0)
        pexc = (run + cs - cc).astype(jnp.float32)
        mf = cc.astype(jnp.float32)
        acc = acc + ss * ((N - 1.0) - pexc - 0.5 * (mf - 1.0))
        run = run + jnp.sum(cc)
        return run, acc
    _, acc = lax.fori_loop(0, BPT // L, scanb,
                           (base, jnp.zeros((L,), jnp.float32)))
    tf_v[...] = acc
    pltpu.sync_copy(tf_v, par_s.at[pl.ds(s * L, L)])
    plsc.subcore_barrier()

    @pl.when(s == 0)
    def _():
        pltpu.sync_copy(par_s, para_v)
        pltpu.sync_copy(unc_s, upa_v)
        numer = jnp.zeros((L,), jnp.float32)
        usum = jnp.zeros((L,), jnp.float32)
        for t in range(NS):
            numer = numer + para_v[pl.ds(t * L, L)]
            usum = usum + upa_v[pl.ds(t * L, L)]
        nsc = jnp.sum(numer)
        usc = jnp.sum(usum)
        res_v[pl.ds(0, L)] = jnp.full((L,), 1.0, jnp.float32) * nsc
        res_v[pl.ds(L, L)] = jnp.full((L,), 1.0, jnp.float32) * usc
        pltpu.sync_copy(res_v, out_h.at[pl.ds(c * 2 * L, 2 * L)])


def kernel(pred, target, uncertainty_map):
    o = _fused(pred, target, uncertainty_map)
    numer = o[0] + o[2 * L]
    usum = o[L] + o[3 * L]
    return numer * (1.0 / DENOM) * (1.0 + usum * (1.0 / N))


# speedup vs baseline: 1.1480x; 1.1261x over previous
import functools

import jax
import jax.numpy as jnp
from jax import lax
from jax.experimental import pallas as pl
from jax.experimental.pallas import tpu as pltpu
from jax.experimental.pallas import tpu_sc as plsc

N = 262144
NC = 2
NS = 16
L = 16
NW = NC * NS
EPW = N // NW
ZB = 32
SHIFT = 16
NBINS = 34816
BPT = NBINS // NS
CHUNK = 128
NCHUNK = EPW // CHUNK
NWAVES = 8
CPW = NCHUNK // NWAVES
UNROLL = 8
HNB = NBINS // 2
HBPT = HNB // NS
DENOM = 34359607296.0

_mesh = plsc.VectorSubcoreMesh(core_axis_name="c", subcore_axis_name="s")
_params = pltpu.CompilerParams(needs_layout_passes=False)


@functools.partial(
    pl.kernel,
    out_type=(
        jax.ShapeDtypeStruct((NC * NBINS,), jnp.int32),
        jax.ShapeDtypeStruct((NC * NBINS,), jnp.float32),
        jax.ShapeDtypeStruct((NW * L,), jnp.float32),
    ),
    mesh=_mesh,
    compiler_params=_params,
    scratch_types=[
        pltpu.VMEM((EPW,), jnp.float32),
        pltpu.VMEM((EPW,), jnp.float32),
        pltpu.VMEM((EPW,), jnp.float32),
        pltpu.VMEM((NCHUNK, CHUNK), jnp.int32),
        pltpu.VMEM((NCHUNK, CHUNK), jnp.float32),
        pltpu.VMEM((NCHUNK, CHUNK), jnp.int32),
        pltpu.VMEM((BPT,), jnp.int32),
        pltpu.VMEM((BPT,), jnp.float32),
        pltpu.VMEM((L,), jnp.float32),
        pltpu.VMEM_SHARED((NBINS,), jnp.int32),
        pltpu.VMEM_SHARED((NBINS,), jnp.float32),
        pltpu.SemaphoreType.DMA,
        pltpu.SemaphoreType.DMA,
    ],
)
def _pass1(pred_h, targ_h, unc_h, cnt_h, sum_h, uncp_h,
           pred_v, targ_v, unc_v, key_v, val_v, one_v,
           zi_v, zf_v, up_v, cnt_s, sum_s, scat_sem, in_sem):
    c = lax.axis_index("c")
    s = lax.axis_index("s")
    wid = c * NS + s
    base = wid * EPW

    in_descs = [
        pltpu.async_copy(pred_h.at[pl.ds(base, EPW)], pred_v, in_sem),
        pltpu.async_copy(targ_h.at[pl.ds(base, EPW)], targ_v, in_sem),
        pltpu.async_copy(unc_h.at[pl.ds(base, EPW)], unc_v, in_sem),
    ]

    def zfill(i, _):
        zi_v[pl.ds(i * L, L)] = jnp.zeros((L,), jnp.int32)
        zf_v[pl.ds(i * L, L)] = jnp.zeros((L,), jnp.float32)
        return 0
    lax.fori_loop(0, BPT // L, zfill, 0)
    pltpu.sync_copy(zi_v, cnt_s.at[pl.ds(s * BPT, BPT)])
    pltpu.sync_copy(zf_v, sum_s.at[pl.ds(s * BPT, BPT)])
    for d in in_descs:
        d.wait()

    shift = jnp.full((L,), SHIFT, jnp.int32)
    ones_i = jnp.ones((L,), jnp.int32)
    zbin = jnp.full((L,), 1, jnp.int32) * s

    plsc.subcore_barrier()

    def compute_wave(w, acc):
        def body(j, acc):
            for k in range(UNROLL):
                i0 = (j * UNROLL + k) * L
                p = pred_v[pl.ds(i0, L)]
                t = targ_v[pl.ds(i0, L)]
                u = unc_v[pl.ds(i0, L)]
                sgn = 2.0 * t - 1.0
                v = jnp.maximum(1.0 - p * sgn, 0.0)
                bits = lax.bitcast_convert_type(v, jnp.int32)
                kk = lax.shift_right_logical(bits, shift) + ZB
                kk = jnp.where(v > 0.0, kk, zbin)
                cj = (j * UNROLL + k) // (CHUNK // L)
                ck = ((j * UNROLL + k) % (CHUNK // L)) * L
                key_v[cj, pl.ds(ck, L)] = kk
                val_v[cj, pl.ds(ck, L)] = v
                one_v[cj, pl.ds(ck, L)] = ones_i
                acc = acc + u
            return acc
        lo = w * CPW * (CHUNK // L // UNROLL) * UNROLL
        return lax.fori_loop(w * CPW * CHUNK // (L * UNROLL),
                             (w + 1) * CPW * CHUNK // (L * UNROLL),
                             body, acc)

    def fire_wave(w):
        descs = []
        for j in range(w * CPW, (w + 1) * CPW):
            descs.append(pltpu.async_copy(
                val_v.at[j], sum_s.at[key_v.at[j]], scat_sem, add=True))
            descs.append(pltpu.async_copy(
                one_v.at[j], cnt_s.at[key_v.at[j]], scat_sem, add=True))
        return descs

    acc = jnp.zeros((L,), jnp.float32)
    pending = []
    for w in range(NWAVES):
        acc = compute_wave(w, acc)
        for d in pending:
            d.wait()
        pending = fire_wave(w)
    for d in pending:
        d.wait()
    up_v[...] = acc

    plsc.subcore_barrier()

    pltpu.sync_copy(cnt_s.at[pl.ds(s * BPT, BPT)],
                    cnt_h.at[pl.ds(c * NBINS + s * BPT, BPT)])
    pltpu.sync_copy(sum_s.at[pl.ds(s * BPT, BPT)],
                    sum_h.at[pl.ds(c * NBINS + s * BPT, BPT)])
    pltpu.sync_copy(up_v, uncp_h.at[pl.ds(wid * L, L)])


@functools.partial(
    pl.kernel,
    out_type=jax.ShapeDtypeStruct((4 * L,), jnp.float32),
    mesh=_mesh,
    compiler_params=_params,
    scratch_types=[
        pltpu.VMEM((HBPT,), jnp.int32),
        pltpu.VMEM((HBPT,), jnp.int32),
        pltpu.VMEM((HBPT,), jnp.float32),
        pltpu.VMEM((HBPT,), jnp.float32),
        pltpu.VMEM((L,), jnp.int32),
        pltpu.VMEM((NS * L,), jnp.int32),
        pltpu.VMEM((L,), jnp.float32),
        pltpu.VMEM((NS * L,), jnp.float32),
        pltpu.VMEM((NS * L,), jnp.float32),
        pltpu.VMEM((2 * L,), jnp.float32),
        pltpu.VMEM_SHARED((NS * L,), jnp.int32),
        pltpu.VMEM_SHARED((NS * L,), jnp.float32),
    ],
)
def _pass2(cnt_h, sum_h, uncp_h, out_h,
           ca_v, cb_v, sa_v, sb_v, tot_v, tota_v, par_v, para_v,
           unc_v, res_v, tot_s, par_s):
    c = lax.axis_index("c")
    s = lax.axis_index("s")
    hb = c * HNB + s * HBPT

    pltpu.sync_copy(cnt_h.at[pl.ds(hb, HBPT)], ca_v)
    pltpu.sync_copy(cnt_h.at[pl.ds(NBINS + hb, HBPT)], cb_v)
    pltpu.sync_copy(sum_h.at[pl.ds(hb, HBPT)], sa_v)
    pltpu.sync_copy(sum_h.at[pl.ds(NBINS + hb, HBPT)], sb_v)

    def totb(i, acc):
        return acc + ca_v[pl.ds(i * L, L)] + cb_v[pl.ds(i * L, L)]
    tot = lax.fori_loop(0, HBPT // L, totb, jnp.zeros((L,), jnp.int32))
    tot_v[...] = tot
    pltpu.sync_copy(tot_v, tot_s.at[pl.ds(s * L, L)])
    plsc.subcore_barrier()
    pltpu.sync_copy(tot_s, tota_v)

    tsum = jnp.int32(0)
    base = jnp.int32(0)
    for t in range(NS):
        rowsum = jnp.sum(tota_v[pl.ds(t * L, L)])
        tsum = tsum + rowsum
        base = base + jnp.where(jnp.int32(t) < s, rowsum, jnp.int32(0))
    base = base + jnp.where(c == 0, jnp.int32(0), jnp.int32(N) - tsum)

    def scanb(i, carry):
        run, acc = carry
        cc = ca_v[pl.ds(i * L, L)] + cb_v[pl.ds(i * L, L)]
        ss = sa_v[pl.ds(i * L, L)] + sb_v[pl.ds(i * L, L)]
        cs = lax.cumsum(cc, axis=0)
        pexc = (run + cs - cc).astype(jnp.float32)
        mf = cc.astype(jnp.float32)
        acc = acc + ss * ((N - 1.0) - pexc - 0.5 * (mf - 1.0))
        run = run + jnp.sum(cc)
        return run, acc
    _, acc = lax.fori_loop(0, HBPT // L, scanb,
                           (base, jnp.zeros((L,), jnp.float32)))
    par_v[...] = acc
    pltpu.sync_copy(par_v, par_s.at[pl.ds(s * L, L)])
    plsc.subcore_barrier()

    @pl.when(s == 0)
    def _():
        pltpu.sync_copy(par_s, para_v)
        pltpu.sync_copy(uncp_h.at[pl.ds(c * NS * L, NS * L)], unc_v)
        numer = jnp.zeros((L,), jnp.float32)
        usum = jnp.zeros((L,), jnp.float32)
        for t in range(NS):
            numer = numer + para_v[pl.ds(t * L, L)]
            usum = usum + unc_v[pl.ds(t * L, L)]
        nsc = jnp.sum(numer)
        usc = jnp.sum(usum)
        res_v[pl.ds(0, L)] = jnp.full((L,), 1.0, jnp.float32) * nsc
        res_v[pl.ds(L, L)] = jnp.full((L,), 1.0, jnp.float32) * usc
        pltpu.sync_copy(res_v, out_h.at[pl.ds(c * 2 * L, 2 * L)])


def kernel(pred, target, uncertainty_map):
    cnt, sm, uncp = _pass1(pred, target, uncertainty_map)
    o = _pass2(cnt, sm, uncp)
    numer = o[0] + o[2 * L]
    usum = o[L] + o[3 * L]
    return numer * (1.0 / DENOM) * (1.0 + usum * (1.0 / N))


# speedup vs baseline: 1.2475x; 1.0867x over previous
import functools

import jax
import jax.numpy as jnp
from jax import lax
from jax.experimental import pallas as pl
from jax.experimental.pallas import tpu as pltpu
from jax.experimental.pallas import tpu_sc as plsc

N = 262144
NC = 2
NS = 16
L = 16
NW = NC * NS
EPW = N // NW
ZB = 32
SHIFT = 16
NBINS = 34816
BPT = NBINS // NS
CHUNK = 128
NCHUNK = EPW // CHUNK
NWAVES = 8
CPW = NCHUNK // NWAVES
UNROLL = 8
DENOM = 34359607296.0

_mesh = plsc.VectorSubcoreMesh(core_axis_name="c", subcore_axis_name="s")
_params = pltpu.CompilerParams(needs_layout_passes=False)


@functools.partial(
    pl.kernel,
    out_type=(
        jax.ShapeDtypeStruct((NC * NBINS,), jnp.int32),
        jax.ShapeDtypeStruct((NC * NBINS,), jnp.float32),
        jax.ShapeDtypeStruct((NW * L,), jnp.float32),
    ),
    mesh=_mesh,
    compiler_params=_params,
    scratch_types=[
        pltpu.VMEM((EPW,), jnp.float32),
        pltpu.VMEM((EPW,), jnp.float32),
        pltpu.VMEM((EPW,), jnp.float32),
        pltpu.VMEM((NCHUNK, CHUNK), jnp.int32),
        pltpu.VMEM((NCHUNK, CHUNK), jnp.float32),
        pltpu.VMEM((NCHUNK, CHUNK), jnp.int32),
        pltpu.VMEM((BPT,), jnp.int32),
        pltpu.VMEM((BPT,), jnp.float32),
        pltpu.VMEM((L,), jnp.float32),
        pltpu.VMEM_SHARED((NBINS,), jnp.int32),
        pltpu.VMEM_SHARED((NBINS,), jnp.float32),
        pltpu.SemaphoreType.DMA,
        pltpu.SemaphoreType.DMA,
    ],
)
def _pass1(pred_h, targ_h, unc_h, cnt_h, sum_h, uncp_h,
           pred_v, targ_v, unc_v, key_v, val_v, one_v,
           zi_v, zf_v, up_v, cnt_s, sum_s, scat_sem, in_sem):
    c = lax.axis_index("c")
    s = lax.axis_index("s")
    wid = c * NS + s
    base = wid * EPW

    in_descs = [
        pltpu.async_copy(pred_h.at[pl.ds(base, EPW)], pred_v, in_sem),
        pltpu.async_copy(targ_h.at[pl.ds(base, EPW)], targ_v, in_sem),
        pltpu.async_copy(unc_h.at[pl.ds(base, EPW)], unc_v, in_sem),
    ]

    def zfill(i, _):
        zi_v[pl.ds(i * L, L)] = jnp.zeros((L,), jnp.int32)
        zf_v[pl.ds(i * L, L)] = jnp.zeros((L,), jnp.float32)
        return 0
    lax.fori_loop(0, BPT // L, zfill, 0)
    pltpu.sync_copy(zi_v, cnt_s.at[pl.ds(s * BPT, BPT)])
    pltpu.sync_copy(zf_v, sum_s.at[pl.ds(s * BPT, BPT)])
    for d in in_descs:
        d.wait()

    shift = jnp.full((L,), SHIFT, jnp.int32)
    ones_i = jnp.ones((L,), jnp.int32)
    zbin = jnp.full((L,), 1, jnp.int32) * s

    plsc.subcore_barrier()

    def compute_wave(w, acc):
        def body(j, acc):
            for k in range(UNROLL):
                i0 = (j * UNROLL + k) * L
                p = pred_v[pl.ds(i0, L)]
                t = targ_v[pl.ds(i0, L)]
                u = unc_v[pl.ds(i0, L)]
                sgn = 2.0 * t - 1.0
                v = jnp.maximum(1.0 - p * sgn, 0.0)
                bits = lax.bitcast_convert_type(v, jnp.int32)
                kk = lax.shift_right_logical(bits, shift) + ZB
                kk = jnp.where(v > 0.0, kk, zbin)
                cj = (j * UNROLL + k) // (CHUNK // L)
                ck = ((j * UNROLL + k) % (CHUNK // L)) * L
                key_v[cj, pl.ds(ck, L)] = kk
                val_v[cj, pl.ds(ck, L)] = v
                one_v[cj, pl.ds(ck, L)] = ones_i
                acc = acc + u
            return acc
        lo = w * CPW * (CHUNK // L // UNROLL) * UNROLL
        return lax.fori_loop(w * CPW * CHUNK // (L * UNROLL),
                             (w + 1) * CPW * CHUNK // (L * UNROLL),
                             body, acc)

    def fire_wave(w):
        descs = []
        for j in range(w * CPW, (w + 1) * CPW):
            descs.append(pltpu.async_copy(
                val_v.at[j], sum_s.at[key_v.at[j]], scat_sem, add=True))
            descs.append(pltpu.async_copy(
                one_v.at[j], cnt_s.at[key_v.at[j]], scat_sem, add=True))
        return descs

    acc = jnp.zeros((L,), jnp.float32)
    pending = []
    for w in range(NWAVES):
        acc = compute_wave(w, acc)
        for d in pending:
            d.wait()
        pending = fire_wave(w)
    for d in pending:
        d.wait()
    up_v[...] = acc

    plsc.subcore_barrier()

    pltpu.sync_copy(cnt_s.at[pl.ds(s * BPT, BPT)],
                    cnt_h.at[pl.ds(c * NBINS + s * BPT, BPT)])
    pltpu.sync_copy(sum_s.at[pl.ds(s * BPT, BPT)],
                    sum_h.at[pl.ds(c * NBINS + s * BPT, BPT)])
    pltpu.sync_copy(up_v, uncp_h.at[pl.ds(wid * L, L)])


@functools.partial(
    pl.kernel,
    out_type=jax.ShapeDtypeStruct((L,), jnp.float32),
    mesh=_mesh,
    compiler_params=_params,
    scratch_types=[
        pltpu.VMEM((BPT,), jnp.int32),
        pltpu.VMEM((BPT,), jnp.int32),
        pltpu.VMEM((BPT,), jnp.float32),
        pltpu.VMEM((BPT,), jnp.float32),
        pltpu.VMEM((L,), jnp.int32),
        pltpu.VMEM((NS * L,), jnp.int32),
        pltpu.VMEM((L,), jnp.float32),
        pltpu.VMEM((NS * L,), jnp.float32),
        pltpu.VMEM((NW * L,), jnp.float32),
        pltpu.VMEM((L,), jnp.float32),
        pltpu.VMEM_SHARED((NS * L,), jnp.int32),
        pltpu.VMEM_SHARED((NS * L,), jnp.float32),
    ],
)
def _pass2(cnt_h, sum_h, uncp_h, out_h,
           ca_v, cb_v, sa_v, sb_v, tot_v, tota_v, par_v, para_v,
           unc_v, res_v, tot_s, par_s):
    c = lax.axis_index("c")
    s = lax.axis_index("s")

    @pl.when(c == 0)
    def _():
        pltpu.sync_copy(cnt_h.at[pl.ds(s * BPT, BPT)], ca_v)
        pltpu.sync_copy(cnt_h.at[pl.ds(NBINS + s * BPT, BPT)], cb_v)
        pltpu.sync_copy(sum_h.at[pl.ds(s * BPT, BPT)], sa_v)
        pltpu.sync_copy(sum_h.at[pl.ds(NBINS + s * BPT, BPT)], sb_v)

        def totb(i, acc):
            return acc + ca_v[pl.ds(i * L, L)] + cb_v[pl.ds(i * L, L)]
        tot = lax.fori_loop(0, BPT // L, totb, jnp.zeros((L,), jnp.int32))
        tot_v[...] = tot
        pltpu.sync_copy(tot_v, tot_s.at[pl.ds(s * L, L)])
        plsc.subcore_barrier()
        pltpu.sync_copy(tot_s, tota_v)

        base = jnp.int32(0)
        for t in range(NS):
            rowsum = jnp.sum(tota_v[pl.ds(t * L, L)])
            base = base + jnp.where(jnp.int32(t) < s, rowsum, jnp.int32(0))

        def scanb(i, carry):
            run, acc = carry
            cc = ca_v[pl.ds(i * L, L)] + cb_v[pl.ds(i * L, L)]
            ss = sa_v[pl.ds(i * L, L)] + sb_v[pl.ds(i * L, L)]
            cs = lax.cumsum(cc, axis=0)
            pexc = (run + cs - cc).astype(jnp.float32)
            mf = cc.astype(jnp.float32)
            acc = acc + ss * ((N - 1.0) - pexc - 0.5 * (mf - 1.0))
            run = run + jnp.sum(cc)
            return run, acc
        _, acc = lax.fori_loop(0, BPT // L, scanb,
                               (base, jnp.zeros((L,), jnp.float32)))
        par_v[...] = acc
        pltpu.sync_copy(par_v, par_s.at[pl.ds(s * L, L)])
        plsc.subcore_barrier()

        @pl.when(s == 0)
        def _():
            pltpu.sync_copy(par_s, para_v)
            pltpu.sync_copy(uncp_h, unc_v)
            numer = jnp.zeros((L,), jnp.float32)
            for t in range(NS):
                numer = numer + para_v[pl.ds(t * L, L)]
            usum = jnp.zeros((L,), jnp.float32)
            for t in range(NW):
                usum = usum + unc_v[pl.ds(t * L, L)]
            nsc = jnp.sum(numer)
            usc = jnp.sum(usum)
            res = nsc * (1.0 / DENOM) * (1.0 + usc * (1.0 / N))
            res_v[...] = jnp.full((L,), 1.0, jnp.float32) * res
            pltpu.sync_copy(res_v, out_h)


def kernel(pred, target, uncertainty_map):
    cnt, sm, uncp = _pass1(pred, target, uncertainty_map)
    out = _pass2(cnt, sm, uncp)
    return out[0]
